# baseline (speedup 1.0000x reference)
import functools

import jax
import jax.numpy as jnp
import numpy as np
from jax import lax
from jax.experimental import pallas as pl
from jax.experimental.pallas import tpu as pltpu
from jax.experimental.pallas import tpu_sc as plsc

BPE = 43
PAD = 48
D = 320
NCHUNK = 20
BATCH = 1024
NROWS = NCHUNK * BATCH
NW = 32
ROWS_PER_W = NROWS // NW
GROUPS_PER_IT = 2
IDX_PER_IT = GROUPS_PER_IT * PAD
NIT = ROWS_PER_W // GROUPS_PER_IT
STAGE_ROWS = 16
IT_PER_BLOCK = STAGE_ROWS // GROUPS_PER_IT
NBLOCKS = NIT // IT_PER_BLOCK
NCOL = D // 16
INV = np.float32(1.0 / BPE)


def _sc_body(table_hbm, idx_hbm, out_hbm,
             idx_v, buf0, buf1, stage0, stage1,
             gsem0, gsem1, osem0, osem1):
    wid = lax.axis_index("s") * 2 + lax.axis_index("c")
    idx_base = pl.multiple_of(wid * (ROWS_PER_W * PAD), 8)
    row_base = wid * ROWS_PER_W

    pltpu.sync_copy(idx_hbm.at[pl.ds(idx_base, ROWS_PER_W * PAD)], idx_v)

    bufs = (buf0, buf1)
    gsems = (gsem0, gsem1)
    stages = (stage0, stage1)
    osems = (osem0, osem1)

    def gather(it, buf, sem):
        for g in range(GROUPS_PER_IT):
            vecs = [idx_v[pl.ds(it * IDX_PER_IT + g * PAD + v * 16, 16)]
                    for v in range(PAD // 16)]
            for j in range(BPE):
                row = vecs[j // 16][j % 16]
                pltpu.async_copy(table_hbm.at[pl.ds(row, 1)],
                                 buf.at[pl.ds(g * PAD + j, 1)], sem)

    def drain(buf, sem):
        pltpu.make_async_copy(
            table_hbm.at[pl.ds(0, GROUPS_PER_IT * BPE)],
            buf.at[pl.ds(0, GROUPS_PER_IT * BPE)], sem).wait()

    gather(0, buf0, gsem0)
    gather(1, buf1, gsem1)

    def reduce_group(buf, rbase):
        def body(j, accs):
            return tuple(acc + buf[rbase + j, pl.ds(c * 16, 16)]
                         for c, acc in enumerate(accs))
        zero = jnp.zeros((16,), jnp.float32)
        return lax.fori_loop(0, BPE, body, (zero,) * NCOL)

    def block_body(t):
        for ob in range(2):
            block = t + ob
            stage = stages[ob]
            osem = osems[ob]

            @pl.when(block >= 2)
            def _wait_prev_out():
                pltpu.make_async_copy(
                    stage, out_hbm.at[pl.ds(row_base, STAGE_ROWS)], osem
                ).wait()

            for k in range(IT_PER_BLOCK):
                gb = k % 2
                buf = bufs[gb]
                it = block * IT_PER_BLOCK + k
                drain(buf, gsems[gb])
                for g in range(GROUPS_PER_IT):
                    accs = reduce_group(buf, g * PAD)
                    for c in range(NCOL):
                        stage[2 * k + g, pl.ds(c * 16, 16)] = accs[c] * INV

                @pl.when(it < NIT - 2)
                def _next_gather():
                    gather(it + 2, buf, gsems[gb])

            out_off = row_base + block * STAGE_ROWS
            pltpu.async_copy(stage,
                             out_hbm.at[pl.ds(out_off, STAGE_ROWS)], osem)

    pl.loop(0, NBLOCKS, step=2)(block_body)

    for ob in range(2):
        pltpu.make_async_copy(
            stages[ob], out_hbm.at[pl.ds(row_base, STAGE_ROWS)], osems[ob]
        ).wait()


@jax.jit
def kernel(tokens, table):
    tok = tokens.reshape(NCHUNK, BPE, BATCH)
    tok = jnp.swapaxes(tok, 1, 2)
    idx = jnp.pad(tok, ((0, 0), (0, 0), (0, PAD - BPE)))
    idx_flat = idx.reshape(NROWS * PAD)

    mesh = plsc.VectorSubcoreMesh(core_axis_name="c", subcore_axis_name="s")
    sc = pl.kernel(
        _sc_body,
        out_type=jax.ShapeDtypeStruct((NROWS, D), jnp.float32),
        mesh=mesh,
        compiler_params=pltpu.CompilerParams(use_tc_tiling_on_sc=False),
        scratch_types=[
            pltpu.VMEM((ROWS_PER_W * PAD,), jnp.int32),
            pltpu.VMEM((IDX_PER_IT, D), jnp.float32),
            pltpu.VMEM((IDX_PER_IT, D), jnp.float32),
            pltpu.VMEM((STAGE_ROWS, D), jnp.float32),
            pltpu.VMEM((STAGE_ROWS, D), jnp.float32),
            pltpu.SemaphoreType.DMA,
            pltpu.SemaphoreType.DMA,
            pltpu.SemaphoreType.DMA,
            pltpu.SemaphoreType.DMA,
        ],
    )
    out = sc(table, idx_flat)
    return out.reshape(NCHUNK, BATCH, D)
